# baseline (speedup 1.0000x reference)
import functools

import jax
import jax.numpy as jnp
from jax import lax
from jax.experimental import pallas as pl
from jax.experimental.pallas import tpu as pltpu
from jax.experimental.pallas import tpu_sc as plsc

_D = 32
_B = 16384
_S = 50
_SP = 56
_LANES = 128

_NW = 32
_BW = 256
_NCH = _B // _BW
_MAIN_UNITS = 6 * _NCH
_TAIL_UNITS = _NCH


def _body(tab_hbm, ids_hbm, out_hbm, idsv, idxb, ev, sem0, sem1, osem):
    wid = lax.axis_index("s") * 2 + lax.axis_index("c")
    sems = (sem0, sem1)

    def run_unit(u, nsl):
        blk = u // _NCH
        s0 = blk * 8
        b0 = (u % _NCH) * _BW
        pltpu.sync_copy(ids_hbm.at[pl.ds(s0, 8), pl.ds(b0, _BW)], idsv)

        def wait_out():
            pltpu.make_async_copy(
                ev.at[0], out_hbm.at[pl.ds(0, _BW)], osem).wait()

        for sl in range(nsl + 1):
            p = sl % 2
            if sl < nsl:
                if sl >= 2:
                    wait_out()
                def idx_build(i, c3, sl=sl, p=p):
                    idxb[p, pl.ds(i * 16, 16)] = lax.shift_right_logical(
                        idsv[sl, pl.ds(i * 16, 16)].astype(jnp.int32), 2)
                    return c3

                lax.fori_loop(0, _BW // 16, idx_build, 0)
                for t in range(_BW // 128):
                    pltpu.async_copy(
                        tab_hbm.at[idxb.at[p, pl.ds(t * 128, 128)]],
                        ev.at[p, pl.ds(t * 128, 128), :], sems[p])
            if sl >= 1:
                q = (sl - 1) % 2
                for t in range(_BW // 128):
                    pltpu.make_async_copy(
                        tab_hbm.at[idxb.at[q, pl.ds(0, 128)]],
                        ev.at[q, pl.ds(t * 128, 128), :], sems[q]).wait()
                row0 = (s0 + (sl - 1)) * _B + b0
                pltpu.async_copy(ev.at[q], out_hbm.at[pl.ds(row0, _BW)],
                                 osem)
        wait_out()
        if nsl >= 2:
            wait_out()

    def main_unit(k, carry):
        run_unit(wid + k * _NW, 8)
        return carry

    def tail_unit(k, carry):
        run_unit(6 * _NCH + wid + k * _NW, 2)
        return carry

    lax.fori_loop(0, _MAIN_UNITS // _NW, main_unit, 0)
    lax.fori_loop(0, _TAIL_UNITS // _NW, tail_unit, 0)


@functools.partial(
    pl.kernel,
    mesh=plsc.VectorSubcoreMesh(core_axis_name="c", subcore_axis_name="s"),
    out_type=jax.ShapeDtypeStruct((_S * _B, _LANES), jnp.float32),
    scratch_types=[
        pltpu.VMEM((8, _BW), jnp.float32),
        pltpu.VMEM((2, _BW), jnp.int32),
        pltpu.VMEM((2, _BW, _LANES), jnp.float32),
        pltpu.SemaphoreType.DMA,
        pltpu.SemaphoreType.DMA,
        pltpu.SemaphoreType.DMA,
    ],
)
def _sc_embed(tab_hbm, ids_hbm, out_hbm, idsv, idxb, ev, sem0, sem1, osem):
    _body(tab_hbm, ids_hbm, out_hbm, idsv, idxb, ev, sem0, sem1, osem)


def kernel(x, embed_weight):
    tabv = embed_weight.reshape(250000, _LANES)
    idsp = jnp.pad(x[:, :, 0], ((0, 0), (0, _SP - _S))).T
    embf = _sc_embed(tabv, idsp)
    sel = jnp.bitwise_and(idsp[:_S].reshape(_S * _B).astype(jnp.int32), 3)
    oh = (sel[:, None] == jnp.arange(4, dtype=jnp.int32)[None, :])
    g = embf.reshape(_S * _B, 4, _D)
    emb_f = jnp.sum(g * oh[:, :, None].astype(jnp.float32), axis=1)
    emb = emb_f.reshape(_S, _B, _D).transpose(1, 0, 2)
    return jnp.concatenate([emb, x[:, :, 1:]], axis=-1)


# speedup vs baseline: 1.6308x; 1.6308x over previous
import functools

import jax
import jax.numpy as jnp
from jax import lax
from jax.experimental import pallas as pl
from jax.experimental.pallas import tpu as pltpu
from jax.experimental.pallas import tpu_sc as plsc

_D = 32
_B = 16384
_S = 50
_SP = 56
_LANES = 128

_NW = 32
_BW = 256
_NCH = _B // _BW
_MAIN_UNITS = 6 * _NCH
_TAIL_UNITS = _NCH


def _body(tab_hbm, ids_hbm, out_hbm, idsv, idxb, ev, sem0, sem1, osem):
    wid = lax.axis_index("s") * 2 + lax.axis_index("c")
    sems = (sem0, sem1)

    def run_unit(u, nsl):
        blk = u // _NCH
        s0 = blk * 8
        b0 = (u % _NCH) * _BW
        pltpu.sync_copy(ids_hbm.at[pl.ds(s0, 8), pl.ds(b0, _BW)], idsv)

        def wait_out():
            pltpu.make_async_copy(
                ev.at[0], out_hbm.at[pl.ds(0, _BW)], osem).wait()

        for sl in range(nsl + 1):
            p = sl % 2
            if sl < nsl:
                if sl >= 2:
                    wait_out()
                def idx_build(i, c3, sl=sl, p=p):
                    idxb[p, pl.ds(i * 16, 16)] = idsv[
                        sl, pl.ds(i * 16, 16)].astype(jnp.int32)
                    return c3

                lax.fori_loop(0, _BW // 16, idx_build, 0)
                for t in range(_BW // 128):
                    pltpu.async_copy(
                        tab_hbm.at[idxb.at[p, pl.ds(t * 128, 128)]],
                        ev.at[p, pl.ds(t * 128, 128), :], sems[p])
            if sl >= 1:
                q = (sl - 1) % 2
                for t in range(_BW // 128):
                    pltpu.make_async_copy(
                        tab_hbm.at[idxb.at[q, pl.ds(0, 128)]],
                        ev.at[q, pl.ds(t * 128, 128), :], sems[q]).wait()
                row0 = (s0 + (sl - 1)) * _B + b0
                pltpu.async_copy(ev.at[q], out_hbm.at[pl.ds(row0, _BW)],
                                 osem)
        wait_out()
        if nsl >= 2:
            wait_out()

    def main_unit(k, carry):
        run_unit(wid + k * _NW, 8)
        return carry

    def tail_unit(k, carry):
        run_unit(6 * _NCH + wid + k * _NW, 2)
        return carry

    lax.fori_loop(0, _MAIN_UNITS // _NW, main_unit, 0)
    lax.fori_loop(0, _TAIL_UNITS // _NW, tail_unit, 0)


@functools.partial(
    pl.kernel,
    mesh=plsc.VectorSubcoreMesh(core_axis_name="c", subcore_axis_name="s"),
    out_type=jax.ShapeDtypeStruct((_S * _B, _LANES), jnp.float32),
    scratch_types=[
        pltpu.VMEM((8, _BW), jnp.float32),
        pltpu.VMEM((2, _BW), jnp.int32),
        pltpu.VMEM((2, _BW, _LANES), jnp.float32),
        pltpu.SemaphoreType.DMA,
        pltpu.SemaphoreType.DMA,
        pltpu.SemaphoreType.DMA,
    ],
)
def _sc_embed(tab_hbm, ids_hbm, out_hbm, idsv, idxb, ev, sem0, sem1, osem):
    _body(tab_hbm, ids_hbm, out_hbm, idsv, idxb, ev, sem0, sem1, osem)


def kernel(x, embed_weight):
    tabp = jnp.pad(embed_weight, ((0, 0), (0, _LANES - _D)))
    idsp = jnp.pad(x[:, :, 0], ((0, 0), (0, _SP - _S))).T
    embf = _sc_embed(tabp, idsp)
    emb = embf.reshape(_S, _B, _LANES)[:, :, :_D].transpose(1, 0, 2)
    return jnp.concatenate([emb, x[:, :, 1:]], axis=-1)


# speedup vs baseline: 1.6311x; 1.0002x over previous
import functools

import jax
import jax.numpy as jnp
from jax import lax
from jax.experimental import pallas as pl
from jax.experimental.pallas import tpu as pltpu
from jax.experimental.pallas import tpu_sc as plsc

_D = 32
_B = 16384
_S = 50
_SP = 56
_LANES = 128

_NW = 32
_BW = 256
_NCH = _B // _BW
_MAIN_UNITS = 6 * _NCH
_TAIL_UNITS = _NCH


def _body(tab_hbm, ids_hbm, out_hbm, idsv, idxb, ev, sem0, sem1, osem):
    wid = lax.axis_index("s") * 2 + lax.axis_index("c")
    sems = (sem0, sem1)

    def run_unit(u, nsl):
        blk = u // _NCH
        s0 = blk * 8
        b0 = (u % _NCH) * _BW
        pltpu.sync_copy(ids_hbm.at[pl.ds(s0, 8), pl.ds(b0, _BW)], idsv)

        def wait_out():
            pltpu.make_async_copy(
                ev.at[0], out_hbm.at[pl.ds(0, _BW)], osem).wait()

        for sl in range(nsl + 1):
            p = sl % 2
            if sl < nsl:
                if sl >= 2:
                    wait_out()
                def idx_build(i, c3, sl=sl, p=p):
                    idxb[p, pl.ds(i * 16, 16)] = idsv[
                        sl, pl.ds(i * 16, 16)].astype(jnp.int32)
                    return c3

                lax.fori_loop(0, _BW // 16, idx_build, 0)
                for t in range(_BW // 128):
                    pltpu.async_copy(
                        tab_hbm.at[idxb.at[p, pl.ds(t * 128, 128)]],
                        ev.at[p, pl.ds(t * 128, 128), :], sems[p])
            if sl >= 1:
                q = (sl - 1) % 2
                for t in range(_BW // 128):
                    pltpu.make_async_copy(
                        tab_hbm.at[idxb.at[q, pl.ds(0, 128)]],
                        ev.at[q, pl.ds(t * 128, 128), :], sems[q]).wait()
                row0 = (s0 + (sl - 1)) * _B + b0
                pltpu.async_copy(ev.at[q], out_hbm.at[pl.ds(row0, _BW)],
                                 osem)
        wait_out()
        if nsl >= 2:
            wait_out()

    def main_unit(k, carry):
        run_unit(wid + k * _NW, 8)
        return carry

    def tail_unit(k, carry):
        run_unit(6 * _NCH + wid + k * _NW, 2)
        return carry

    lax.fori_loop(0, _MAIN_UNITS // _NW, main_unit, 0)
    lax.fori_loop(0, _TAIL_UNITS // _NW, tail_unit, 0)


@functools.partial(
    pl.kernel,
    mesh=plsc.VectorSubcoreMesh(core_axis_name="c", subcore_axis_name="s"),
    out_type=jax.ShapeDtypeStruct((_S * _B, _LANES), jnp.float32),
    scratch_types=[
        pltpu.VMEM((8, _BW), jnp.float32),
        pltpu.VMEM((2, _BW), jnp.int32),
        pltpu.VMEM((2, _BW, _LANES), jnp.float32),
        pltpu.SemaphoreType.DMA,
        pltpu.SemaphoreType.DMA,
        pltpu.SemaphoreType.DMA,
    ],
)
def _sc_embed(tab_hbm, ids_hbm, out_hbm, idsv, idxb, ev, sem0, sem1, osem):
    _body(tab_hbm, ids_hbm, out_hbm, idsv, idxb, ev, sem0, sem1, osem)


def kernel(x, embed_weight):
    tabp = jnp.concatenate(
        [embed_weight,
         jnp.zeros((embed_weight.shape[0], _LANES - _D), jnp.float32)],
        axis=1)
    idsp = jnp.pad(x[:, :, 0], ((0, 0), (0, _SP - _S))).T
    embf = _sc_embed(tabp, idsp)
    emb = embf.reshape(_S, _B, _LANES)[:, :, :_D].transpose(1, 0, 2)
    return jnp.concatenate([emb, x[:, :, 1:]], axis=-1)


# speedup vs baseline: 1.6322x; 1.0006x over previous
import functools

import jax
import jax.numpy as jnp
from jax import lax
from jax.experimental import pallas as pl
from jax.experimental.pallas import tpu as pltpu
from jax.experimental.pallas import tpu_sc as plsc

_D = 32
_B = 16384
_S = 50
_SP = 56
_LANES = 128

_NW = 32
_BW = 256
_NCH = _B // _BW
_MAIN_UNITS = 6 * _NCH
_TAIL_UNITS = _NCH


def _body(tab_hbm, ids_hbm, out_hbm, idsv, idxb, ev,
          sem0, sem1, sem2, osem0, osem1):
    wid = lax.axis_index("s") * 2 + lax.axis_index("c")
    sems = (sem0, sem1, sem2)
    osems = (osem0, osem1)

    def run_unit(u, nsl):
        blk = u // _NCH
        s0 = blk * 8
        b0 = (u % _NCH) * _BW
        pltpu.sync_copy(ids_hbm.at[pl.ds(s0, 8), pl.ds(b0, _BW)], idsv)

        def wait_out(par):
            pltpu.make_async_copy(
                ev.at[pl.ds(0, _BW)], out_hbm.at[pl.ds(0, _BW)],
                osems[par]).wait()

        for sl in range(nsl + 1):
            p = sl % 3
            if sl < nsl:
                if sl >= 3:
                    wait_out((sl - 2) % 2)

                def idx_build(i, c3, sl=sl, p=p):
                    idxb[pl.ds(p * _BW + i * 16, 16)] = idsv[
                        sl, pl.ds(i * 16, 16)].astype(jnp.int32)
                    return c3

                lax.fori_loop(0, _BW // 16, idx_build, 0)
                for t in range(_BW // 128):
                    pltpu.async_copy(
                        tab_hbm.at[idxb.at[pl.ds(p * _BW + t * 128, 128)]],
                        ev.at[pl.ds(p * _BW + t * 128, 128), :], sems[p])
            if sl >= 1:
                q = (sl - 1) % 3
                for t in range(_BW // 128):
                    pltpu.make_async_copy(
                        tab_hbm.at[idxb.at[pl.ds(0, 128)]],
                        ev.at[pl.ds(q * _BW + t * 128, 128), :],
                        sems[q]).wait()
                row0 = (s0 + (sl - 1)) * _B + b0
                pltpu.async_copy(ev.at[pl.ds(q * _BW, _BW)],
                                 out_hbm.at[pl.ds(row0, _BW)],
                                 osems[sl % 2])
        for s in range(max(1, nsl - 2), nsl + 1):
            wait_out(s % 2)

    def main_unit(k, carry):
        run_unit(wid + k * _NW, 8)
        return carry

    def tail_unit(k, carry):
        run_unit(6 * _NCH + wid + k * _NW, 2)
        return carry

    lax.fori_loop(0, _MAIN_UNITS // _NW, main_unit, 0)
    lax.fori_loop(0, _TAIL_UNITS // _NW, tail_unit, 0)


@functools.partial(
    pl.kernel,
    mesh=plsc.VectorSubcoreMesh(core_axis_name="c", subcore_axis_name="s"),
    out_type=jax.ShapeDtypeStruct((_S * _B, _LANES), jnp.float32),
    scratch_types=[
        pltpu.VMEM((8, _BW), jnp.float32),
        pltpu.VMEM((3 * _BW,), jnp.int32),
        pltpu.VMEM((3 * _BW, _LANES), jnp.float32),
        pltpu.SemaphoreType.DMA,
        pltpu.SemaphoreType.DMA,
        pltpu.SemaphoreType.DMA,
        pltpu.SemaphoreType.DMA,
        pltpu.SemaphoreType.DMA,
    ],
)
def _sc_embed(tab_hbm, ids_hbm, out_hbm, idsv, idxb, ev,
              sem0, sem1, sem2, osem0, osem1):
    _body(tab_hbm, ids_hbm, out_hbm, idsv, idxb, ev,
          sem0, sem1, sem2, osem0, osem1)


def kernel(x, embed_weight):
    tabp = jnp.concatenate(
        [embed_weight,
         jnp.zeros((embed_weight.shape[0], _LANES - _D), jnp.float32)],
        axis=1)
    idsp = jnp.pad(x[:, :, 0], ((0, 0), (0, _SP - _S))).T
    embf = _sc_embed(tabp, idsp)
    emb = embf.reshape(_S, _B, _LANES)[:, :, :_D].transpose(1, 0, 2)
    return jnp.concatenate([emb, x[:, :, 1:]], axis=-1)
